# Initial kernel scaffold; baseline (speedup 1.0000x reference)
import jax
import jax.numpy as jnp
from jax.experimental import pallas as pl


def kernel(inputs, embeddings):
    raise NotImplementedError("write your pallas kernel here")


# speedup vs baseline: 2.2794x; 2.2794x over previous
import functools

import jax
import jax.numpy as jnp
from jax import lax
from jax.experimental import pallas as pl
from jax.experimental.pallas import tpu as pltpu
from jax.experimental.pallas import tpu_sc as plsc


def _transpose_body(e_ref, t_ref):
    t_ref[...] = e_ref[...].T


def _transpose(embeddings, chunk=2048):
    d, v = embeddings.shape
    return pl.pallas_call(
        _transpose_body,
        grid=(pl.cdiv(v, chunk),),
        in_specs=[pl.BlockSpec((d, chunk), lambda i: (0, i))],
        out_specs=pl.BlockSpec((chunk, d), lambda i: (i, 0)),
        out_shape=jax.ShapeDtypeStruct((v, d), jnp.float32),
    )(embeddings)


def _make_gather(V, D, B):
    info = plsc.get_sparse_core_info()
    nw = info.num_cores * info.num_subcores
    b_per_w = B // nw
    assert B % nw == 0
    CH = 128
    n_ch = b_per_w // CH
    assert b_per_w % CH == 0
    mesh = plsc.VectorSubcoreMesh(core_axis_name="c", subcore_axis_name="s")

    @functools.partial(
        pl.kernel,
        mesh=mesh,
        out_type=jax.ShapeDtypeStruct((B, D), jnp.float32),
        scratch_types=[
            pltpu.VMEM((n_ch, CH), jnp.int32),
            pltpu.VMEM((CH, D), jnp.float32),
            pltpu.SemaphoreType.DMA,
        ],
    )
    def k(table_hbm, idx_hbm, out_hbm, idx_v, rows_v, sem):
        wid = lax.axis_index("s") * info.num_cores + lax.axis_index("c")
        base = wid * b_per_w
        pltpu.sync_copy(idx_hbm.at[wid], idx_v)

        def body(c, carry):
            pltpu.async_copy(table_hbm.at[idx_v.at[c]], rows_v, sem).wait()
            pltpu.sync_copy(rows_v, out_hbm.at[pl.ds(base + c * CH, CH)])
            return carry

        lax.fori_loop(0, n_ch, body, 0)

    return k, nw, n_ch, CH


def kernel(inputs, embeddings):
    d, v = embeddings.shape
    b, s = inputs.shape
    n = b * s
    table = _transpose(embeddings)

    gather, nw, n_ch, ch = _make_gather(v, d, n)
    idx = inputs.reshape(nw, n_ch, ch).astype(jnp.int32)
    out = gather(table, idx)
    return out.reshape(b, s, d)


# speedup vs baseline: 7.5409x; 3.3082x over previous
import functools

import jax
import jax.numpy as jnp
from jax import lax
from jax.experimental import pallas as pl
from jax.experimental.pallas import tpu as pltpu
from jax.experimental.pallas import tpu_sc as plsc


def _make_gather(D, B):
    info = plsc.get_sparse_core_info()
    nw = info.num_cores * info.num_subcores
    b_per_w = B // nw
    assert B % nw == 0
    CH = 128
    n_ch = b_per_w // CH
    assert b_per_w % CH == 0
    mesh = plsc.VectorSubcoreMesh(core_axis_name="c", subcore_axis_name="s")

    @functools.partial(
        pl.kernel,
        mesh=mesh,
        out_type=jax.ShapeDtypeStruct((B, D), jnp.float32),
        scratch_types=[
            pltpu.VMEM((n_ch, CH), jnp.int32),
            pltpu.VMEM((CH, D), jnp.float32),
            pltpu.SemaphoreType.DMA,
        ],
    )
    def k(table_hbm, idx_hbm, out_hbm, idx_v, rows_v, sem):
        wid = lax.axis_index("s") * info.num_cores + lax.axis_index("c")
        base = wid * b_per_w
        pltpu.sync_copy(idx_hbm.at[wid], idx_v)

        def body(c, carry):
            pltpu.async_copy(table_hbm.at[idx_v.at[c]], rows_v, sem).wait()
            pltpu.sync_copy(rows_v, out_hbm.at[pl.ds(base + c * CH, CH)])
            return carry

        lax.fori_loop(0, n_ch, body, 0)

    return k, nw, n_ch, CH


def kernel(inputs, embeddings):
    d, v = embeddings.shape
    b, s = inputs.shape
    n = b * s
    table = jnp.transpose(embeddings)

    gather, nw, n_ch, ch = _make_gather(d, n)
    idx = jnp.transpose(inputs).reshape(nw, n_ch, ch).astype(jnp.int32)
    out = gather(table, idx)
    return out.reshape(s, b, d).transpose(1, 0, 2)


# speedup vs baseline: 10.3922x; 1.3781x over previous
import functools

import jax
import jax.numpy as jnp
from jax import lax
from jax.experimental import pallas as pl
from jax.experimental.pallas import tpu as pltpu
from jax.experimental.pallas import tpu_sc as plsc


def _make_gather(D, B):
    info = plsc.get_sparse_core_info()
    nw = info.num_cores * info.num_subcores
    b_per_w = B // nw
    assert B % nw == 0
    CH = 128
    n_ch = b_per_w // CH
    assert b_per_w % CH == 0
    mesh = plsc.VectorSubcoreMesh(core_axis_name="c", subcore_axis_name="s")

    assert n_ch % 2 == 0

    @functools.partial(
        pl.kernel,
        mesh=mesh,
        out_type=jax.ShapeDtypeStruct((B, D), jnp.float32),
        scratch_types=[
            pltpu.VMEM((n_ch, CH), jnp.int32),
            pltpu.VMEM((CH, D), jnp.float32),
            pltpu.VMEM((CH, D), jnp.float32),
            pltpu.SemaphoreType.DMA,
            pltpu.SemaphoreType.DMA,
        ],
    )
    def k(table_hbm, idx_hbm, out_hbm, idx_v, rows0, rows1, sem0, sem1):
        wid = lax.axis_index("s") * info.num_cores + lax.axis_index("c")
        base = wid * b_per_w
        pltpu.sync_copy(idx_hbm.at[wid], idx_v)

        def gather(c, buf, sem):
            return pltpu.async_copy(table_hbm.at[idx_v.at[c]], buf, sem)

        def wait(buf, sem):
            pltpu.make_async_copy(table_hbm.at[idx_v.at[0]], buf, sem).wait()

        def store(c, buf):
            pltpu.sync_copy(buf, out_hbm.at[pl.ds(base + c * CH, CH)])

        gather(0, rows0, sem0)

        def body(i, carry):
            g = 2 * i
            gather(g + 1, rows1, sem1)
            wait(rows0, sem0)
            store(g, rows0)

            @pl.when(g + 2 < n_ch)
            def _():
                gather(g + 2, rows0, sem0)

            wait(rows1, sem1)
            store(g + 1, rows1)
            return carry

        lax.fori_loop(0, n_ch // 2, body, 0)

    return k, nw, n_ch, CH


def kernel(inputs, embeddings):
    d, v = embeddings.shape
    b, s = inputs.shape
    n = b * s
    table = jnp.transpose(embeddings)

    gather, nw, n_ch, ch = _make_gather(d, n)
    idx = jnp.transpose(inputs).reshape(nw, n_ch, ch).astype(jnp.int32)
    out = gather(table, idx)
    return out.reshape(s, b, d).transpose(1, 0, 2)
